# baseline (speedup 1.0000x reference)
import functools

import jax
import jax.numpy as jnp
from jax import lax
from jax.experimental import pallas as pl
from jax.experimental.pallas import tpu as pltpu
from jax.experimental.pallas import tpu_sc as plsc

B = 16384
N_OBJ = 64
N_ACT = 8
P_SPACE = 0.1
EXIST_THR = 0.8

NC, NS, L = 2, 16, 16
NW = NC * NS
ROWS_PER_W = B // NW
NB = 16
TILES = ROWS_PER_W // NB
GPR = N_OBJ
GAT = NB * GPR


def _sc_body(x_hbm, act_hbm, mask_hbm, ap_hbm, pv_hbm,
             idx_v, gat_v, mask_v, mexp_v, act_v, sat_v, ap_v, pv_v, sem):
    wid = lax.axis_index("s") * NC + lax.axis_index("c")
    base_row = wid * ROWS_PER_W

    pltpu.sync_copy(mask_hbm, mask_v)
    pltpu.sync_copy(act_hbm, act_v)

    iota = lax.iota(jnp.int32, L)
    i4 = iota * 4
    r64 = iota * 64

    for m in range(N_OBJ // L):
        chunk = mask_v[pl.ds(m * L, L)]
        for j in range(L):
            mexp_v[pl.ds((m * L + j) * L, L)] = jnp.broadcast_to(
                chunk[j], (L,))

    def tile(t, carry):
        b0 = base_row + t * NB
        gbase = b0 * 256
        for k in range(GAT // L):
            l_loc, m = k // 4, k % 4
            cvec = i4 + (gbase + 256 * l_loc + 65 * m)
            idx_v[k // 8, pl.ds((k % 8) * L, L)] = cvec
        copies = []
        for j in range(8):
            copies.append(
                pltpu.async_copy(x_hbm.at[idx_v.at[j]],
                                 gat_v.at[pl.ds(j * 128, 128)], sem))
        for c in copies:
            c.wait()

        zero = jnp.zeros((L,), jnp.int32)
        a_val = plsc.load_gather(gat_v, [r64, zero])
        b_val = plsc.load_gather(gat_v, [r64 + 1, zero])
        p = jnp.abs(a_val - b_val)
        acc = p < P_SPACE
        for i in range(N_OBJ):
            rows = r64 + i
            col = jnp.full((L,), i % 16, jnp.int32)
            diag = plsc.load_gather(gat_v, [rows, col])
            m_i = mexp_v[pl.ds(i * L, L)] > 0
            acc = acc & ((diag > EXIST_THR) == m_i)
        satf = jnp.where(acc, 1.0, 0.0).astype(jnp.float32)

        pv_v[pl.ds(t * NB, NB)] = p
        sat_v[...] = satf
        act = act_v[...]
        an = act / (act + 1e-20)
        half = (iota >= 8).astype(jnp.int32)
        for pair in range(NB // 2):
            sel = half + 2 * pair
            ap_v[pl.ds(t * NB * N_ACT + pair * L, L)] = (
                plsc.load_gather(sat_v, [sel]) * an)
        return carry

    lax.fori_loop(0, TILES, tile, 0, unroll=False)

    pltpu.sync_copy(pv_v, pv_hbm.at[pl.ds(base_row, ROWS_PER_W)])
    pltpu.sync_copy(ap_v, ap_hbm.at[pl.ds(base_row * N_ACT,
                                          ROWS_PER_W * N_ACT)])


@jax.jit
def _run(x2, act2, mask_i32):
    mesh = plsc.VectorSubcoreMesh(core_axis_name="c", subcore_axis_name="s")
    f = functools.partial(
        pl.kernel,
        mesh=mesh,
        compiler_params=pltpu.CompilerParams(needs_layout_passes=False,
                                             use_tc_tiling_on_sc=False),
        out_type=[
            jax.ShapeDtypeStruct((B * N_ACT,), jnp.float32),
            jax.ShapeDtypeStruct((B,), jnp.float32),
        ],
        scratch_types=[
            pltpu.VMEM((8, 128), jnp.int32),
            pltpu.VMEM((GAT, L), jnp.float32),
            pltpu.VMEM((N_OBJ,), jnp.int32),
            pltpu.VMEM((N_OBJ * L,), jnp.int32),
            pltpu.VMEM((L,), jnp.float32),
            pltpu.VMEM((L,), jnp.float32),
            pltpu.VMEM((ROWS_PER_W * N_ACT,), jnp.float32),
            pltpu.VMEM((ROWS_PER_W,), jnp.float32),
            pltpu.SemaphoreType.DMA,
        ],
    )(_sc_body)
    return f(x2, act2, mask_i32)


def kernel(x, action, mask):
    x2 = x.reshape(B * 256, 16)
    act2 = jnp.concatenate([action, action]).astype(jnp.float32)
    mask_i32 = mask.astype(jnp.int32)
    ap_flat, pv = _run(x2, act2, mask_i32)
    return (ap_flat.reshape(B, N_ACT), pv)


# speedup vs baseline: 1.0444x; 1.0444x over previous
import functools

import jax
import jax.numpy as jnp
from jax import lax
from jax.experimental import pallas as pl
from jax.experimental.pallas import tpu as pltpu
from jax.experimental.pallas import tpu_sc as plsc

B = 16384
N_OBJ = 64
N_ACT = 8
P_SPACE = 0.1
EXIST_THR = 0.8

NC, NS, L = 2, 16, 16
NW = NC * NS
ROWS_PER_W = B // NW
NB = 16
TILES = ROWS_PER_W // NB
GPR = N_OBJ
GAT = NB * GPR
NCOPY = GAT // 128


def _sc_body(x_hbm, act_hbm, mask_hbm, ap_hbm, pv_hbm,
             idx_a, idx_b, gat_a, gat_b, mask_v, mexp_v, act_v, sat_v,
             ap_v, pv_v, sem_a, sem_b):
    wid = lax.axis_index("s") * NC + lax.axis_index("c")
    base_row = wid * ROWS_PER_W

    pltpu.sync_copy(mask_hbm, mask_v)
    pltpu.sync_copy(act_hbm, act_v)

    iota = lax.iota(jnp.int32, L)
    i4 = iota * 4
    r64 = iota * 64

    for m in range(N_OBJ // L):
        chunk = mask_v[pl.ds(m * L, L)]
        for j in range(L):
            mexp_v[pl.ds((m * L + j) * L, L)] = jnp.broadcast_to(
                chunk[j], (L,))

    act = act_v[...]
    an = act / (act + 1e-20)
    half = (iota >= 8).astype(jnp.int32)
    zero = jnp.zeros((L,), jnp.int32)

    def fire(t, idx_v, gat_v, sem):
        gbase = (base_row + t * NB) * 256
        for k in range(GAT // L):
            l_loc, m = k // 4, k % 4
            cvec = i4 + (gbase + 256 * l_loc + 65 * m)
            idx_v[k // 8, pl.ds((k % 8) * L, L)] = cvec
        for j in range(NCOPY):
            pltpu.async_copy(x_hbm.at[idx_v.at[j]],
                             gat_v.at[pl.ds(j * 128, 128)], sem)

    def drain(idx_v, gat_v, sem):
        for j in range(NCOPY):
            pltpu.make_async_copy(x_hbm.at[idx_v.at[j]],
                                  gat_v.at[pl.ds(j * 128, 128)], sem).wait()

    def compute(t, gat_v):
        a_val = plsc.load_gather(gat_v, [r64, zero])
        b_val = plsc.load_gather(gat_v, [r64 + 1, zero])
        p = jnp.abs(a_val - b_val)
        acc = p < P_SPACE
        for i in range(N_OBJ):
            rows = r64 + i
            col = jnp.full((L,), i % 16, jnp.int32)
            diag = plsc.load_gather(gat_v, [rows, col])
            m_i = mexp_v[pl.ds(i * L, L)] > 0
            acc = acc & ((diag > EXIST_THR) == m_i)
        satf = jnp.where(acc, 1.0, 0.0).astype(jnp.float32)

        pv_v[pl.ds(t * NB, NB)] = p
        sat_v[...] = satf
        for pair in range(NB // 2):
            sel = half + 2 * pair
            ap_v[pl.ds(t * NB * N_ACT + pair * L, L)] = (
                plsc.load_gather(sat_v, [sel]) * an)

    fire(0, idx_a, gat_a, sem_a)

    def step(s, carry):
        ta = 2 * s
        tb = 2 * s + 1
        fire(tb, idx_b, gat_b, sem_b)
        drain(idx_a, gat_a, sem_a)
        compute(ta, gat_a)
        ta_next = jnp.minimum(ta + 2, TILES - 1)
        fire(ta_next, idx_a, gat_a, sem_a)
        drain(idx_b, gat_b, sem_b)
        compute(tb, gat_b)
        return carry

    lax.fori_loop(0, TILES // 2, step, 0, unroll=False)
    drain(idx_a, gat_a, sem_a)

    pltpu.sync_copy(pv_v, pv_hbm.at[pl.ds(base_row, ROWS_PER_W)])
    pltpu.sync_copy(ap_v, ap_hbm.at[pl.ds(base_row * N_ACT,
                                          ROWS_PER_W * N_ACT)])


@jax.jit
def _run(x2, act2, mask_i32):
    mesh = plsc.VectorSubcoreMesh(core_axis_name="c", subcore_axis_name="s")
    f = functools.partial(
        pl.kernel,
        mesh=mesh,
        compiler_params=pltpu.CompilerParams(needs_layout_passes=False,
                                             use_tc_tiling_on_sc=False),
        out_type=[
            jax.ShapeDtypeStruct((B * N_ACT,), jnp.float32),
            jax.ShapeDtypeStruct((B,), jnp.float32),
        ],
        scratch_types=[
            pltpu.VMEM((NCOPY, 128), jnp.int32),
            pltpu.VMEM((NCOPY, 128), jnp.int32),
            pltpu.VMEM((GAT, L), jnp.float32),
            pltpu.VMEM((GAT, L), jnp.float32),
            pltpu.VMEM((N_OBJ,), jnp.int32),
            pltpu.VMEM((N_OBJ * L,), jnp.int32),
            pltpu.VMEM((L,), jnp.float32),
            pltpu.VMEM((L,), jnp.float32),
            pltpu.VMEM((ROWS_PER_W * N_ACT,), jnp.float32),
            pltpu.VMEM((ROWS_PER_W,), jnp.float32),
            pltpu.SemaphoreType.DMA,
            pltpu.SemaphoreType.DMA,
        ],
    )(_sc_body)
    return f(x2, act2, mask_i32)


def kernel(x, action, mask):
    x2 = x.reshape(B * 256, 16)
    act2 = jnp.concatenate([action, action]).astype(jnp.float32)
    mask_i32 = mask.astype(jnp.int32)
    ap_flat, pv = _run(x2, act2, mask_i32)
    return (ap_flat.reshape(B, N_ACT), pv)
